# baseline (speedup 1.0000x reference)
import functools

import jax
import jax.numpy as jnp
import numpy as np
from jax import lax
from jax.experimental import pallas as pl
from jax.experimental.pallas import tpu as pltpu
from jax.experimental.pallas import tpu_sc as plsc

N = 10000
NP = 10240
E = 320000
EP = 327680
D = 128
G = 64
NW = 32
EROWS = EP // 128
ERPT = EROWS // NW
SLAB = NP // 16
DUMP = NP - 1
NPH = NP // 2
ACCR = NPH + 128
LDUMP = NPH
ERPS = EROWS // 16
ASLAB = ACCR // 16

_mesh = plsc.VectorSubcoreMesh(core_axis_name="c", subcore_axis_name="s")

_TRIU1 = np.triu(np.ones((G, G), np.float32), 1)


CL = 10752


@functools.partial(
    pl.kernel,
    out_type=(jax.ShapeDtypeStruct((NW, NP // 128, 128), jnp.float32),
              jax.ShapeDtypeStruct((NW, 1, 128), jnp.float32),
              jax.ShapeDtypeStruct((2 * NW * CL,), jnp.int32),
              jax.ShapeDtypeStruct((2 * NW * CL,), jnp.int32),
              jax.ShapeDtypeStruct((NW * 16,), jnp.int32)),
    mesh=_mesh,
    scratch_types=[
        pltpu.VMEM((ERPT, 128), jnp.int32),
        pltpu.VMEM((ERPT, 128), jnp.int32),
        pltpu.VMEM((CL,), jnp.int32),
        pltpu.VMEM((CL,), jnp.int32),
        pltpu.VMEM((CL,), jnp.int32),
        pltpu.VMEM((CL,), jnp.int32),
        pltpu.VMEM((NP // NW,), jnp.int32),
        pltpu.VMEM((1, 128), jnp.float32),
        pltpu.VMEM((NP // 128, 128), jnp.float32),
    ],
    compiler_params=pltpu.CompilerParams(needs_layout_passes=False),
)
def _hist_kernel(edges_hbm, batch_hbm, deg_out, cnt_out,
                 csrc_out, cdst_out, ccnt_out,
                 src_vm, dst_vm, csa_vm, cda_vm, csb_vm, cdb_vm, bat_vm,
                 cnt_vm, deg_vm):
    c = lax.axis_index("c")
    s = lax.axis_index("s")
    wid = s * 2 + c
    ones = jnp.ones((16,), jnp.float32)
    zeros = jnp.zeros((16,), jnp.float32)
    c127 = jnp.full((16,), 127, jnp.int32)
    zi = jnp.zeros((16,), jnp.int32)

    @pl.loop(0, NP // 128)
    def _(i):
        for k in range(8):
            deg_vm[i, pl.ds(k * 16, 16)] = zeros

    for k in range(8):
        cnt_vm[0, pl.ds(k * 16, 16)] = zeros

    pltpu.sync_copy(edges_hbm.at[0, pl.ds(wid * ERPT, ERPT)], src_vm)
    pltpu.sync_copy(edges_hbm.at[1, pl.ds(wid * ERPT, ERPT)], dst_vm)
    pltpu.sync_copy(batch_hbm.at[pl.ds(wid * (NP // NW), NP // NW)], bat_vm)

    @pl.loop(0, ERPT)
    def _(j):
        for k in range(8):
            idx = src_vm[j, pl.ds(k * 16, 16)]
            plsc.addupdate_scatter(deg_vm, [idx >> 7, idx & c127], ones)

    @pl.loop(0, NP // NW // 16)
    def _(j):
        idx = bat_vm[pl.ds(j * 16, 16)]
        plsc.addupdate_scatter(cnt_vm, [zi, idx], ones)

    nph16 = jnp.full((16,), NPH, jnp.int32)
    ld16 = jnp.full((16,), LDUMP, jnp.int32)
    zi16 = jnp.zeros((16,), jnp.int32)
    lane_iota = lax.iota(jnp.int32, 16)
    full_mask = lane_iota >= zi16

    def _scalar(v):
        return jnp.sum(jnp.where(lane_iota == zi16, v, zi16))

    @pl.loop(0, ERPT, init_carry=(jnp.int32(0), jnp.int32(0)))
    def _compact(j, carry):
        off_a, off_b = carry
        for k in range(8):
            sl = pl.ds(k * 16, 16)
            s16 = src_vm[j, sl]
            d16 = dst_vm[j, sl]
            m_a = d16 < nph16
            plsc.store_compressed(csa_vm.at[pl.ds(off_a, 16)], s16, mask=m_a)
            plsc.store_compressed(cda_vm.at[pl.ds(off_a, 16)], d16, mask=m_a)
            m_b = d16 >= nph16
            plsc.store_compressed(csb_vm.at[pl.ds(off_b, 16)], s16, mask=m_b)
            plsc.store_compressed(cdb_vm.at[pl.ds(off_b, 16)], d16 - nph16, mask=m_b)
            n_a = _scalar(plsc.all_reduce_population_count(m_a))
            off_a = off_a + n_a
            off_b = off_b + (16 - n_a)
        return off_a, off_b

    off_a, off_b = _compact
    for t in range(24):
        plsc.store_compressed(csa_vm.at[pl.ds(off_a + t * 16, 16)], zi16, mask=full_mask)
        plsc.store_compressed(cda_vm.at[pl.ds(off_a + t * 16, 16)], ld16, mask=full_mask)
        plsc.store_compressed(csb_vm.at[pl.ds(off_b + t * 16, 16)], zi16, mask=full_mask)
        plsc.store_compressed(cdb_vm.at[pl.ds(off_b + t * 16, 16)], ld16, mask=full_mask)
    n2a = ((off_a + 127) // 128 + 2) // 2 * 2
    n2b = ((off_b + 127) // 128 + 2) // 2 * 2
    cntv = jnp.where(lane_iota == zi16, lax.broadcast(n2a, (16,)),
                     jnp.where(lane_iota == jnp.ones((16,), jnp.int32),
                               lax.broadcast(n2b, (16,)), zi16))

    pltpu.sync_copy(csa_vm, csrc_out.at[pl.ds(wid * CL, CL)])
    pltpu.sync_copy(cda_vm, cdst_out.at[pl.ds(wid * CL, CL)])
    pltpu.sync_copy(csb_vm, csrc_out.at[pl.ds((NW + wid) * CL, CL)])
    pltpu.sync_copy(cdb_vm, cdst_out.at[pl.ds((NW + wid) * CL, CL)])
    pltpu.sync_copy(deg_vm, deg_out.at[wid])
    pltpu.sync_copy(cnt_vm, cnt_out.at[wid])
    csa_vm[pl.ds(0, 16)] = cntv
    pltpu.sync_copy(csa_vm.at[pl.ds(0, 16)], ccnt_out.at[pl.ds(wid * 16, 16)])


@functools.partial(
    pl.kernel,
    out_type=jax.ShapeDtypeStruct((2, ACCR, D), jnp.float32),
    mesh=_mesh,
    scratch_types=[
        pltpu.VMEM((CL,), jnp.int32),
        pltpu.VMEM((CL,), jnp.int32),
        pltpu.VMEM((CL // 128, 128), jnp.int32),
        pltpu.VMEM((NW * 16,), jnp.int32),
        pltpu.VMEM((128, D), jnp.float32),
        pltpu.VMEM((128, D), jnp.float32),
        pltpu.SemaphoreType.DMA,
        pltpu.SemaphoreType.DMA,
        pltpu.VMEM_SHARED((ACCR, D), jnp.float32),
    ],
    compiler_params=pltpu.CompilerParams(needs_layout_passes=False),
)
def _scatter_kernel(u_hbm, csrc_hbm, cdst_hbm, ccnt_hbm, out_hbm,
                    src_vm, dst1_vm, dst_vm, cnt_vm, rows_a, rows_b,
                    sem_a, sem_b, acc):
    c = lax.axis_index("c")
    s = lax.axis_index("s")

    zeros = jnp.zeros((16,), jnp.float32)
    zi16 = jnp.zeros((16,), jnp.int32)
    lane_iota = lax.iota(jnp.int32, 16)

    @pl.loop(0, 128)
    def _(r):
        for k in range(8):
            rows_a[r, pl.ds(k * 16, 16)] = zeros

    pltpu.sync_copy(rows_a, acc.at[pl.ds(s * ASLAB, 128)])
    pltpu.sync_copy(rows_a, acc.at[pl.ds(s * ASLAB + 128, 128)])
    pltpu.sync_copy(rows_a.at[pl.ds(0, ASLAB - 256)],
                    acc.at[pl.ds(s * ASLAB + 256, ASLAB - 256)])
    pltpu.sync_copy(ccnt_hbm, cnt_vm)
    plsc.subcore_barrier()

    core_sel = lax.broadcast(c, (16,))
    for blk in range(2):
        b = s * 2 + blk
        pltpu.sync_copy(csrc_hbm.at[pl.ds((c * NW + b) * CL, CL)], src_vm)
        pltpu.sync_copy(cdst_hbm.at[pl.ds((c * NW + b) * CL, CL)], dst1_vm)
        cv = cnt_vm[pl.ds(b * 16, 16)]
        n2 = jnp.sum(jnp.where(lane_iota == core_sel, cv, zi16))

        @pl.loop(0, n2)
        def _(j):
            for k in range(8):
                dst_vm[j, pl.ds(k * 16, 16)] = dst1_vm[pl.ds(j * 128 + k * 16, 16)]

        pltpu.async_copy(u_hbm.at[src_vm.at[pl.ds(0, 128)]], rows_a, sem_a)

        @pl.loop(0, n2 // 2 - 1)
        def _(i):
            j = i * 2
            pltpu.make_async_copy(u_hbm.at[src_vm.at[pl.ds(j * 128, 128)]], rows_a, sem_a).wait()
            pltpu.async_copy(u_hbm.at[src_vm.at[pl.ds((j + 1) * 128, 128)]], rows_b, sem_b)
            pltpu.sync_copy(rows_a, acc.at[dst_vm.at[j]], add=True)
            pltpu.make_async_copy(u_hbm.at[src_vm.at[pl.ds((j + 1) * 128, 128)]], rows_b, sem_b).wait()
            pltpu.async_copy(u_hbm.at[src_vm.at[pl.ds((j + 2) * 128, 128)]], rows_a, sem_a)
            pltpu.sync_copy(rows_b, acc.at[dst_vm.at[j + 1]], add=True)

        jj = n2 - 2
        pltpu.make_async_copy(u_hbm.at[src_vm.at[pl.ds(jj * 128, 128)]], rows_a, sem_a).wait()
        pltpu.async_copy(u_hbm.at[src_vm.at[pl.ds((jj + 1) * 128, 128)]], rows_b, sem_b)
        pltpu.sync_copy(rows_a, acc.at[dst_vm.at[jj]], add=True)
        pltpu.make_async_copy(u_hbm.at[src_vm.at[pl.ds((jj + 1) * 128, 128)]], rows_b, sem_b).wait()
        pltpu.sync_copy(rows_b, acc.at[dst_vm.at[jj + 1]], add=True)

    plsc.subcore_barrier()
    pltpu.sync_copy(acc.at[pl.ds(s * ASLAB, ASLAB)],
                    out_hbm.at[c, pl.ds(s * ASLAB, ASLAB)])


@functools.partial(
    pl.kernel,
    out_type=(jax.ShapeDtypeStruct((G, D), jnp.float32),
              jax.ShapeDtypeStruct((G, D), jnp.float32)),
    mesh=_mesh,
    scratch_types=[
        pltpu.VMEM((2, G), jnp.int32),
        pltpu.VMEM((32, D), jnp.float32),
        pltpu.VMEM((D,), jnp.float32),
        pltpu.VMEM((D,), jnp.float32),
    ],
    compiler_params=pltpu.CompilerParams(needs_layout_passes=False),
)
def _pool_kernel(h_hbm, scb_hbm, sum_out, max_out, scb_vm, rbuf, sacc, macc):
    c = lax.axis_index("c")
    s = lax.axis_index("s")
    wid = s * 2 + c
    pltpu.sync_copy(scb_hbm, scb_vm)
    lane_iota = lax.iota(jnp.int32, 16)
    for gi in range(2):
        g = wid * 2 + gi
        chunk = g // 16
        lane = g % 16
        vst = scb_vm[0, pl.ds(chunk * 16, 16)]
        vct = scb_vm[1, pl.ds(chunk * 16, 16)]
        zero16 = jnp.zeros((16,), jnp.int32)
        st = jnp.sum(jnp.where(lane_iota == lane, vst, zero16))
        ct = jnp.sum(jnp.where(lane_iota == lane, vct, zero16))
        for k in range(8):
            sl = pl.ds(k * 16, 16)
            sacc[sl] = jnp.zeros((16,), jnp.float32)
            macc[sl] = jnp.full((16,), -jnp.inf, jnp.float32)
        b0 = st // 32
        b1 = (st + ct + 31) // 32

        @pl.loop(b0, b1)
        def _(i):
            pltpu.sync_copy(h_hbm.at[pl.ds(i * 32, 32)], rbuf)
            for r in range(32):
                ri = i * 32 + r

                @pl.when((ri >= st) & (ri < st + ct))
                def _():
                    for k in range(8):
                        sl = pl.ds(k * 16, 16)
                        v = rbuf[r, sl]
                        sacc[sl] = sacc[sl] + v
                        macc[sl] = jnp.maximum(macc[sl], v)

        nonempty = lax.broadcast(ct, (16,)) > jnp.zeros((16,), jnp.int32)
        for k in range(8):
            sl = pl.ds(k * 16, 16)
            macc[sl] = jnp.where(nonempty, macc[sl], jnp.zeros((16,), jnp.float32))
        pltpu.sync_copy(sacc, sum_out.at[g])
        pltpu.sync_copy(macc, max_out.at[g])


def _prep_body(dall, x, dinv_out, u0_out):
    deg = jnp.sum(dall[...], axis=0)[:, None]
    dinv = jnp.where(deg > 0.0, lax.rsqrt(deg), 0.0)
    dinv_out[...] = dinv
    u0_out[...] = dinv * x[...]


def _starts_body(cnt, tri, scb_out):
    counts = jnp.sum(cnt[...][:, :G], axis=0).reshape(1, G)
    starts = jnp.dot(counts, tri[...], preferred_element_type=jnp.float32)
    scb_out[...] = jnp.concatenate([starts, counts], axis=0).astype(jnp.int32)


def _mid_body(s0, dinv, u1_out):
    d = dinv[...]
    u1_out[...] = -(d * d) * s0[0]


def _layer_body(h, s0, s1, dinv, w, b, hn_out, un_out):
    d = dinv[...]
    hv = h[...]
    t1 = -d * s0[0]
    t2 = -2.0 * d * s1[0] - hv
    acc = jnp.dot(hv, w[0], preferred_element_type=jnp.float32)
    acc = acc + jnp.dot(t1, w[1], preferred_element_type=jnp.float32)
    acc = acc + jnp.dot(t2, w[2], preferred_element_type=jnp.float32)
    hh = jnp.maximum(acc + b[...], 0.0)
    hn_out[...] = hh
    un_out[...] = d * hh


def _head_body(sums, mx, scb, wfc, bfc, out):
    w = wfc[...]
    counts = scb[1:2, :].astype(jnp.float32)
    bc = jnp.broadcast_to(counts, (G, G))
    ir = lax.broadcasted_iota(jnp.int32, (G, G), 0)
    ic = lax.broadcasted_iota(jnp.int32, (G, G), 1)
    dmat = jnp.where(ir == ic, 1.0 / jnp.maximum(bc, 1.0), 0.0)
    mean = jnp.dot(dmat, sums[...], preferred_element_type=jnp.float32)
    logits = (jnp.dot(mean, w[0:D, :], preferred_element_type=jnp.float32)
              + jnp.dot(mx[...], w[D:2 * D, :], preferred_element_type=jnp.float32)
              + bfc[...])
    m = jnp.max(logits, axis=1, keepdims=True)
    lse = jnp.log(jnp.sum(jnp.exp(logits - m), axis=1, keepdims=True)) + m
    out[...] = logits - lse


_BR = 512
_NBLK = NP // _BR

_row_spec = pl.BlockSpec((_BR, D), lambda i: (i, 0))
_col_spec = pl.BlockSpec((_BR, 1), lambda i: (i, 0))
_pair_spec = pl.BlockSpec((1, _BR, D), lambda i: (i // (NPH // _BR), i % (NPH // _BR), 0))

_prep_call = pl.pallas_call(
    _prep_body,
    grid=(_NBLK,),
    in_specs=[pl.BlockSpec((NW, _BR), lambda i: (0, i)), _row_spec],
    out_specs=[_col_spec, _row_spec],
    out_shape=[jax.ShapeDtypeStruct((NP, 1), jnp.float32),
               jax.ShapeDtypeStruct((NP, D), jnp.float32)],
)

_starts_call = pl.pallas_call(
    _starts_body,
    out_shape=jax.ShapeDtypeStruct((2, G), jnp.int32),
)

_mid_call = pl.pallas_call(
    _mid_body,
    grid=(_NBLK,),
    in_specs=[_pair_spec, _col_spec],
    out_specs=_row_spec,
    out_shape=jax.ShapeDtypeStruct((NP, D), jnp.float32),
)

_layer_call = pl.pallas_call(
    _layer_body,
    grid=(_NBLK,),
    in_specs=[_row_spec, _pair_spec, _pair_spec, _col_spec,
              pl.BlockSpec((3, D, D), lambda i: (0, 0, 0)),
              pl.BlockSpec((1, D), lambda i: (0, 0))],
    out_specs=[_row_spec, _row_spec],
    out_shape=[jax.ShapeDtypeStruct((NP, D), jnp.float32),
               jax.ShapeDtypeStruct((NP, D), jnp.float32)],
)

_head_call = pl.pallas_call(
    _head_body,
    out_shape=jax.ShapeDtypeStruct((G, 6), jnp.float32),
)


def kernel(x, edge_index, batch, W1, b1, W2, b2, W3, b3, W4, b4, Wfc, bfc):
    src = edge_index[0].astype(jnp.int32)
    dst = edge_index[1].astype(jnp.int32)
    pad = jnp.full((EP - E,), DUMP, jnp.int32)
    src2 = jnp.concatenate([src, pad]).reshape(1, EROWS, 128)
    dst2 = jnp.concatenate([dst, pad]).reshape(1, EROWS, 128)
    filler = jnp.full((2, EROWS, 128), DUMP, jnp.int32)
    edges = jnp.concatenate([src2, dst2], axis=0)
    edges = jnp.concatenate([edges, filler], axis=1).reshape(2, 2 * EROWS, 128)
    batchp = jnp.concatenate(
        [batch.astype(jnp.int32), jnp.full((NP - N,), G, jnp.int32)])
    x_pad = jnp.pad(x, ((0, NP - N), (0, 0)))
    tri = jnp.asarray(_TRIU1)

    deg_parts, cnt_parts, csrc, cdst, ccnt = _hist_kernel(edges, batchp)
    dinv, u = _prep_call(deg_parts.reshape(NW, NP), x_pad)
    scb = _starts_call(cnt_parts.reshape(NW, 128), tri)

    h = x_pad
    for (w, b) in ((W1, b1), (W2, b2), (W3, b3), (W4, b4)):
        s0 = _scatter_kernel(u, csrc, cdst, ccnt)
        u1 = _mid_call(s0, dinv)
        s1 = _scatter_kernel(u1, csrc, cdst, ccnt)
        h, u = _layer_call(h, s0, s1, dinv, w, b.reshape(1, D))

    sums, maxs = _pool_kernel(h, scb)
    return _head_call(sums, maxs, scb, Wfc, bfc.reshape(1, 6))


# speedup vs baseline: 1.0350x; 1.0350x over previous
import functools

import jax
import jax.numpy as jnp
import numpy as np
from jax import lax
from jax.experimental import pallas as pl
from jax.experimental.pallas import tpu as pltpu
from jax.experimental.pallas import tpu_sc as plsc

N = 10000
NP = 10240
E = 320000
EP = 327680
D = 128
G = 64
NW = 32
EROWS = EP // 128
ERPT = EROWS // NW
SLAB = NP // 16
DUMP = NP - 1
NPH = NP // 2
ACCR = NPH + 128
LDUMP = NPH
ERPS = EROWS // 16
ASLAB = ACCR // 16

_mesh = plsc.VectorSubcoreMesh(core_axis_name="c", subcore_axis_name="s")

_TRIU1 = np.triu(np.ones((G, G), np.float32), 1)


CL = 10752


@functools.partial(
    pl.kernel,
    out_type=(jax.ShapeDtypeStruct((NW, NP // 128, 128), jnp.float32),
              jax.ShapeDtypeStruct((NW, 1, 128), jnp.float32),
              jax.ShapeDtypeStruct((2, EROWS, 128), jnp.int32)),
    mesh=_mesh,
    scratch_types=[
        pltpu.VMEM((ERPT, 128), jnp.int32),
        pltpu.VMEM((ERPT, 128), jnp.int32),
        pltpu.VMEM((ERPT, 128), jnp.int32),
        pltpu.VMEM((ERPT, 128), jnp.int32),
        pltpu.VMEM((NP // NW,), jnp.int32),
        pltpu.VMEM((1, 128), jnp.float32),
        pltpu.VMEM((NP // 128, 128), jnp.float32),
    ],
    compiler_params=pltpu.CompilerParams(needs_layout_passes=False),
)
def _hist_kernel(edges_hbm, batch_hbm, deg_out, cnt_out, rmp_out,
                 src_vm, dst_vm, rma_vm, rmb_vm, bat_vm,
                 cnt_vm, deg_vm):
    c = lax.axis_index("c")
    s = lax.axis_index("s")
    wid = s * 2 + c
    ones = jnp.ones((16,), jnp.float32)
    zeros = jnp.zeros((16,), jnp.float32)
    c127 = jnp.full((16,), 127, jnp.int32)
    zi = jnp.zeros((16,), jnp.int32)

    @pl.loop(0, NP // 128)
    def _(i):
        for k in range(8):
            deg_vm[i, pl.ds(k * 16, 16)] = zeros

    for k in range(8):
        cnt_vm[0, pl.ds(k * 16, 16)] = zeros

    pltpu.sync_copy(edges_hbm.at[0, pl.ds(wid * ERPT, ERPT)], src_vm)
    pltpu.sync_copy(edges_hbm.at[1, pl.ds(wid * ERPT, ERPT)], dst_vm)
    pltpu.sync_copy(batch_hbm.at[pl.ds(wid * (NP // NW), NP // NW)], bat_vm)

    @pl.loop(0, ERPT)
    def _(j):
        for k in range(8):
            idx = src_vm[j, pl.ds(k * 16, 16)]
            plsc.addupdate_scatter(deg_vm, [idx >> 7, idx & c127], ones)

    @pl.loop(0, NP // NW // 16)
    def _(j):
        idx = bat_vm[pl.ds(j * 16, 16)]
        plsc.addupdate_scatter(cnt_vm, [zi, idx], ones)

    nph16 = jnp.full((16,), NPH, jnp.int32)
    ld16 = jnp.full((16,), LDUMP, jnp.int32)

    @pl.loop(0, ERPT)
    def _(j):
        for k in range(8):
            sl = pl.ds(k * 16, 16)
            d = dst_vm[j, sl]
            rma_vm[j, sl] = jnp.where(d < nph16, d, ld16)
            rmb_vm[j, sl] = jnp.where(d < nph16, ld16, d - nph16)

    pltpu.sync_copy(rma_vm, rmp_out.at[0, pl.ds(wid * ERPT, ERPT)])
    pltpu.sync_copy(rmb_vm, rmp_out.at[1, pl.ds(wid * ERPT, ERPT)])
    pltpu.sync_copy(deg_vm, deg_out.at[wid])
    pltpu.sync_copy(cnt_vm, cnt_out.at[wid])


@functools.partial(
    pl.kernel,
    out_type=jax.ShapeDtypeStruct((2, ACCR, D), jnp.float32),
    mesh=_mesh,
    scratch_types=[
        pltpu.VMEM((ERPS, 128), jnp.int32),
        pltpu.VMEM((ERPS, 128), jnp.int32),
        pltpu.VMEM((128, D), jnp.float32),
        pltpu.VMEM((128, D), jnp.float32),
        pltpu.SemaphoreType.DMA,
        pltpu.SemaphoreType.DMA,
        pltpu.VMEM_SHARED((ACCR, D), jnp.float32),
    ],
)
def _scatter_kernel(u_hbm, edges_hbm, rmp_hbm, out_hbm,
                    src_vm, dst_vm, rows_a, rows_b, sem_a, sem_b, acc):
    c = lax.axis_index("c")
    s = lax.axis_index("s")

    zeros = jnp.zeros((16,), jnp.float32)

    @pl.loop(0, 128)
    def _(r):
        for k in range(8):
            rows_a[r, pl.ds(k * 16, 16)] = zeros

    pltpu.sync_copy(rows_a, acc.at[pl.ds(s * ASLAB, 128)])
    pltpu.sync_copy(rows_a, acc.at[pl.ds(s * ASLAB + 128, 128)])
    pltpu.sync_copy(rows_a.at[pl.ds(0, ASLAB - 256)],
                    acc.at[pl.ds(s * ASLAB + 256, ASLAB - 256)])
    pltpu.sync_copy(edges_hbm.at[0, pl.ds(s * ERPS, ERPS)], src_vm)
    pltpu.sync_copy(rmp_hbm.at[c, pl.ds(s * ERPS, ERPS)], dst_vm)
    pltpu.async_copy(u_hbm.at[src_vm.at[0]], rows_a, sem_a)
    plsc.subcore_barrier()

    @pl.loop(0, ERPS // 2 - 1)
    def _(i):
        j = i * 2
        pltpu.make_async_copy(u_hbm.at[src_vm.at[j]], rows_a, sem_a).wait()
        pltpu.async_copy(u_hbm.at[src_vm.at[j + 1]], rows_b, sem_b)
        pltpu.sync_copy(rows_a, acc.at[dst_vm.at[j]], add=True)
        pltpu.make_async_copy(u_hbm.at[src_vm.at[j + 1]], rows_b, sem_b).wait()
        pltpu.async_copy(u_hbm.at[src_vm.at[j + 2]], rows_a, sem_a)
        pltpu.sync_copy(rows_b, acc.at[dst_vm.at[j + 1]], add=True)

    pltpu.make_async_copy(u_hbm.at[src_vm.at[ERPS - 2]], rows_a, sem_a).wait()
    pltpu.async_copy(u_hbm.at[src_vm.at[ERPS - 1]], rows_b, sem_b)
    pltpu.sync_copy(rows_a, acc.at[dst_vm.at[ERPS - 2]], add=True)
    pltpu.make_async_copy(u_hbm.at[src_vm.at[ERPS - 1]], rows_b, sem_b).wait()
    pltpu.sync_copy(rows_b, acc.at[dst_vm.at[ERPS - 1]], add=True)

    plsc.subcore_barrier()
    pltpu.sync_copy(acc.at[pl.ds(s * ASLAB, ASLAB)],
                    out_hbm.at[c, pl.ds(s * ASLAB, ASLAB)])


@functools.partial(
    pl.kernel,
    out_type=(jax.ShapeDtypeStruct((G, D), jnp.float32),
              jax.ShapeDtypeStruct((G, D), jnp.float32)),
    mesh=_mesh,
    scratch_types=[
        pltpu.VMEM((2, G), jnp.int32),
        pltpu.VMEM((32, D), jnp.float32),
        pltpu.VMEM((D,), jnp.float32),
        pltpu.VMEM((D,), jnp.float32),
    ],
    compiler_params=pltpu.CompilerParams(needs_layout_passes=False),
)
def _pool_kernel(h_hbm, scb_hbm, sum_out, max_out, scb_vm, rbuf, sacc, macc):
    c = lax.axis_index("c")
    s = lax.axis_index("s")
    wid = s * 2 + c
    pltpu.sync_copy(scb_hbm, scb_vm)
    lane_iota = lax.iota(jnp.int32, 16)
    for gi in range(2):
        g = wid * 2 + gi
        chunk = g // 16
        lane = g % 16
        vst = scb_vm[0, pl.ds(chunk * 16, 16)]
        vct = scb_vm[1, pl.ds(chunk * 16, 16)]
        zero16 = jnp.zeros((16,), jnp.int32)
        st = jnp.sum(jnp.where(lane_iota == lane, vst, zero16))
        ct = jnp.sum(jnp.where(lane_iota == lane, vct, zero16))
        for k in range(8):
            sl = pl.ds(k * 16, 16)
            sacc[sl] = jnp.zeros((16,), jnp.float32)
            macc[sl] = jnp.full((16,), -jnp.inf, jnp.float32)
        b0 = st // 32
        b1 = (st + ct + 31) // 32

        @pl.loop(b0, b1)
        def _(i):
            pltpu.sync_copy(h_hbm.at[pl.ds(i * 32, 32)], rbuf)
            for r in range(32):
                ri = i * 32 + r

                @pl.when((ri >= st) & (ri < st + ct))
                def _():
                    for k in range(8):
                        sl = pl.ds(k * 16, 16)
                        v = rbuf[r, sl]
                        sacc[sl] = sacc[sl] + v
                        macc[sl] = jnp.maximum(macc[sl], v)

        nonempty = lax.broadcast(ct, (16,)) > jnp.zeros((16,), jnp.int32)
        for k in range(8):
            sl = pl.ds(k * 16, 16)
            macc[sl] = jnp.where(nonempty, macc[sl], jnp.zeros((16,), jnp.float32))
        pltpu.sync_copy(sacc, sum_out.at[g])
        pltpu.sync_copy(macc, max_out.at[g])


def _prep_body(dall, x, dinv_out, u0_out):
    deg = jnp.sum(dall[...], axis=0)[:, None]
    dinv = jnp.where(deg > 0.0, lax.rsqrt(deg), 0.0)
    dinv_out[...] = dinv
    u0_out[...] = dinv * x[...]


def _starts_body(cnt, tri, scb_out):
    counts = jnp.sum(cnt[...][:, :G], axis=0).reshape(1, G)
    starts = jnp.dot(counts, tri[...], preferred_element_type=jnp.float32)
    scb_out[...] = jnp.concatenate([starts, counts], axis=0).astype(jnp.int32)


def _mid_body(s0, dinv, u1_out):
    d = dinv[...]
    u1_out[...] = -(d * d) * s0[0]


def _layer_body(h, s0, s1, dinv, w, b, hn_out, un_out):
    d = dinv[...]
    hv = h[...]
    t1 = -d * s0[0]
    t2 = -2.0 * d * s1[0] - hv
    acc = jnp.dot(hv, w[0], preferred_element_type=jnp.float32)
    acc = acc + jnp.dot(t1, w[1], preferred_element_type=jnp.float32)
    acc = acc + jnp.dot(t2, w[2], preferred_element_type=jnp.float32)
    hh = jnp.maximum(acc + b[...], 0.0)
    hn_out[...] = hh
    un_out[...] = d * hh


def _head_body(sums, mx, scb, wfc, bfc, out):
    w = wfc[...]
    counts = scb[1:2, :].astype(jnp.float32)
    bc = jnp.broadcast_to(counts, (G, G))
    ir = lax.broadcasted_iota(jnp.int32, (G, G), 0)
    ic = lax.broadcasted_iota(jnp.int32, (G, G), 1)
    dmat = jnp.where(ir == ic, 1.0 / jnp.maximum(bc, 1.0), 0.0)
    mean = jnp.dot(dmat, sums[...], preferred_element_type=jnp.float32)
    logits = (jnp.dot(mean, w[0:D, :], preferred_element_type=jnp.float32)
              + jnp.dot(mx[...], w[D:2 * D, :], preferred_element_type=jnp.float32)
              + bfc[...])
    m = jnp.max(logits, axis=1, keepdims=True)
    lse = jnp.log(jnp.sum(jnp.exp(logits - m), axis=1, keepdims=True)) + m
    out[...] = logits - lse


_BR = 512
_NBLK = NP // _BR

_row_spec = pl.BlockSpec((_BR, D), lambda i: (i, 0))
_col_spec = pl.BlockSpec((_BR, 1), lambda i: (i, 0))
_pair_spec = pl.BlockSpec((1, _BR, D), lambda i: (i // (NPH // _BR), i % (NPH // _BR), 0))

_prep_call = pl.pallas_call(
    _prep_body,
    grid=(_NBLK,),
    in_specs=[pl.BlockSpec((NW, _BR), lambda i: (0, i)), _row_spec],
    out_specs=[_col_spec, _row_spec],
    out_shape=[jax.ShapeDtypeStruct((NP, 1), jnp.float32),
               jax.ShapeDtypeStruct((NP, D), jnp.float32)],
)

_starts_call = pl.pallas_call(
    _starts_body,
    out_shape=jax.ShapeDtypeStruct((2, G), jnp.int32),
)

_mid_call = pl.pallas_call(
    _mid_body,
    grid=(_NBLK,),
    in_specs=[_pair_spec, _col_spec],
    out_specs=_row_spec,
    out_shape=jax.ShapeDtypeStruct((NP, D), jnp.float32),
)

_layer_call = pl.pallas_call(
    _layer_body,
    grid=(_NBLK,),
    in_specs=[_row_spec, _pair_spec, _pair_spec, _col_spec,
              pl.BlockSpec((3, D, D), lambda i: (0, 0, 0)),
              pl.BlockSpec((1, D), lambda i: (0, 0))],
    out_specs=[_row_spec, _row_spec],
    out_shape=[jax.ShapeDtypeStruct((NP, D), jnp.float32),
               jax.ShapeDtypeStruct((NP, D), jnp.float32)],
)

_head_call = pl.pallas_call(
    _head_body,
    out_shape=jax.ShapeDtypeStruct((G, 6), jnp.float32),
)


def kernel(x, edge_index, batch, W1, b1, W2, b2, W3, b3, W4, b4, Wfc, bfc):
    src = edge_index[0].astype(jnp.int32)
    dst = edge_index[1].astype(jnp.int32)
    pad = jnp.full((EP - E,), DUMP, jnp.int32)
    src2 = jnp.concatenate([src, pad]).reshape(1, EROWS, 128)
    dst2 = jnp.concatenate([dst, pad]).reshape(1, EROWS, 128)
    filler = jnp.full((2, EROWS, 128), DUMP, jnp.int32)
    edges = jnp.concatenate([src2, dst2], axis=0)
    edges = jnp.concatenate([edges, filler], axis=1).reshape(2, 2 * EROWS, 128)
    batchp = jnp.concatenate(
        [batch.astype(jnp.int32), jnp.full((NP - N,), G, jnp.int32)])
    x_pad = jnp.pad(x, ((0, NP - N), (0, 0)))
    tri = jnp.asarray(_TRIU1)

    deg_parts, cnt_parts, rmp = _hist_kernel(edges, batchp)
    dinv, u = _prep_call(deg_parts.reshape(NW, NP), x_pad)
    scb = _starts_call(cnt_parts.reshape(NW, 128), tri)

    h = x_pad
    for (w, b) in ((W1, b1), (W2, b2), (W3, b3), (W4, b4)):
        s0 = _scatter_kernel(u, edges, rmp)
        u1 = _mid_call(s0, dinv)
        s1 = _scatter_kernel(u1, edges, rmp)
        h, u = _layer_call(h, s0, s1, dinv, w, b.reshape(1, D))

    sums, maxs = _pool_kernel(h, scb)
    return _head_call(sums, maxs, scb, Wfc, bfc.reshape(1, 6))
